# baseline (speedup 1.0000x reference)
import functools

import jax
import jax.numpy as jnp
from jax import lax
from jax.experimental import pallas as pl
from jax.experimental.pallas import tpu as pltpu
from jax.experimental.pallas import tpu_sc as plsc

_D = 1000
_DP = 1024
_K = 32


@functools.lru_cache(maxsize=None)
def _make_gather(n_tok: int, d: int):
    info = plsc.get_sparse_core_info()
    nc, ns = info.num_cores, info.num_subcores
    nw = nc * ns
    assert n_tok % (nw * _K) == 0
    nchunk = n_tok // (nw * _K)
    mesh = plsc.VectorSubcoreMesh(core_axis_name="c", subcore_axis_name="s")

    @functools.partial(
        pl.kernel,
        out_type=jax.ShapeDtypeStruct((n_tok, d), jnp.float32),
        mesh=mesh,
        scratch_types=[
            pltpu.VMEM((nchunk, _K), jnp.int32),
            pltpu.VMEM((_K, d), jnp.float32),
            pltpu.SemaphoreType.DMA,
        ],
        compiler_params=pltpu.CompilerParams(use_tc_tiling_on_sc=False),
    )
    def gather_kernel(idx_hbm, table_hbm, out_hbm, idx_v, rows_v, gsem):
        wid = lax.axis_index("s") * nc + lax.axis_index("c")
        base = wid * nchunk * _K
        pltpu.sync_copy(idx_hbm.at[wid], idx_v)

        def body(j, carry):
            pltpu.async_copy(table_hbm.at[idx_v.at[j]], rows_v, gsem).wait()
            pltpu.sync_copy(rows_v, out_hbm.at[pl.ds(base + j * _K, _K)])
            return carry

        lax.fori_loop(0, nchunk, body, 0)

    return gather_kernel, nw, nchunk


def kernel(x, table):
    b, l = x.shape
    n_tok = b * l
    gather, nw, nchunk = _make_gather(n_tok, _D)
    idx = x.astype(jnp.int32).reshape(nw, nchunk, _K)
    out = gather(idx, table)
    return out.reshape(b, l, _D)


# speedup vs baseline: 1.0419x; 1.0419x over previous
import functools

import jax
import jax.numpy as jnp
from jax import lax
from jax.experimental import pallas as pl
from jax.experimental.pallas import tpu as pltpu
from jax.experimental.pallas import tpu_sc as plsc

_D = 1000
_K = 40


@functools.lru_cache(maxsize=None)
def _make_gather(n_tok: int, d: int):
    info = plsc.get_sparse_core_info()
    nc, ns = info.num_cores, info.num_subcores
    nw = nc * ns
    assert n_tok % (nw * _K) == 0
    nchunk = n_tok // (nw * _K)
    assert nchunk % 2 == 0
    mesh = plsc.VectorSubcoreMesh(core_axis_name="c", subcore_axis_name="s")

    @functools.partial(
        pl.kernel,
        out_type=jax.ShapeDtypeStruct((n_tok, d), jnp.float32),
        mesh=mesh,
        scratch_types=[
            pltpu.VMEM((nchunk, _K), jnp.int32),
            pltpu.VMEM((_K, d), jnp.float32),
            pltpu.VMEM((_K, d), jnp.float32),
            pltpu.SemaphoreType.DMA,
            pltpu.SemaphoreType.DMA,
            pltpu.SemaphoreType.DMA,
            pltpu.SemaphoreType.DMA,
        ],
        compiler_params=pltpu.CompilerParams(use_tc_tiling_on_sc=False),
    )
    def gather_kernel(idx_hbm, table_hbm, out_hbm,
                      idx_v, buf0, buf1, g0, g1, o0, o1):
        wid = lax.axis_index("s") * nc + lax.axis_index("c")
        base = wid * nchunk * _K
        pltpu.sync_copy(idx_hbm.at[wid], idx_v)

        bufs = (buf0, buf1)
        gsem = (g0, g1)
        osem = (o0, o1)

        def start_g(j, b):
            pltpu.async_copy(table_hbm.at[idx_v.at[j]], bufs[b], gsem[b])

        def wait_g(b):
            pltpu.make_async_copy(table_hbm.at[idx_v.at[0]], bufs[b],
                                  gsem[b]).wait()

        def start_o(j, b):
            pltpu.async_copy(bufs[b], out_hbm.at[pl.ds(base + j * _K, _K)],
                             osem[b])

        def wait_o(b):
            pltpu.make_async_copy(bufs[b], out_hbm.at[pl.ds(base, _K)],
                                  osem[b]).wait()

        start_g(0, 0)
        wait_g(0)
        start_g(1, 1)
        start_o(0, 0)

        def pair(gi, carry):
            j = 2 * gi + 1
            wait_g(1)
            wait_o(0)
            start_g(j + 1, 0)
            start_o(j, 1)
            wait_g(0)
            wait_o(1)
            start_g(j + 2, 1)
            start_o(j + 1, 0)
            return carry

        lax.fori_loop(0, (nchunk - 2) // 2, pair, 0)

        wait_g(1)
        wait_o(0)
        start_o(nchunk - 1, 1)
        wait_o(1)

    return gather_kernel, nw, nchunk


def kernel(x, table):
    b, l = x.shape
    n_tok = b * l
    gather, nw, nchunk = _make_gather(n_tok, _D)
    idx = x.astype(jnp.int32).reshape(nw, nchunk, _K)
    out = gather(idx, table)
    return out.reshape(b, l, _D)


# speedup vs baseline: 1.0443x; 1.0022x over previous
import functools

import jax
import jax.numpy as jnp
from jax import lax
from jax.experimental import pallas as pl
from jax.experimental.pallas import tpu as pltpu
from jax.experimental.pallas import tpu_sc as plsc


@functools.lru_cache(maxsize=None)
def _make_gather(b: int, l: int, d: int):
    info = plsc.get_sparse_core_info()
    nc, ns = info.num_cores, info.num_subcores
    nw = nc * ns
    assert b % nw == 0
    nchunk = b // nw
    assert nchunk % 2 == 0
    mesh = plsc.VectorSubcoreMesh(core_axis_name="c", subcore_axis_name="s")

    @functools.partial(
        pl.kernel,
        out_type=jax.ShapeDtypeStruct((b, l, d), jnp.float32),
        mesh=mesh,
        scratch_types=[
            pltpu.VMEM((nchunk, l), jnp.int32),
            pltpu.VMEM((l, d), jnp.float32),
            pltpu.VMEM((l, d), jnp.float32),
            pltpu.SemaphoreType.DMA,
            pltpu.SemaphoreType.DMA,
            pltpu.SemaphoreType.DMA,
            pltpu.SemaphoreType.DMA,
        ],
        compiler_params=pltpu.CompilerParams(use_tc_tiling_on_sc=False),
    )
    def gather_kernel(idx_hbm, table_hbm, out_hbm,
                      idx_v, buf0, buf1, g0, g1, o0, o1):
        wid = lax.axis_index("s") * nc + lax.axis_index("c")
        base = wid * nchunk
        pltpu.sync_copy(idx_hbm.at[wid], idx_v)

        bufs = (buf0, buf1)
        gsem = (g0, g1)
        osem = (o0, o1)

        def start_g(j, bf):
            pltpu.async_copy(table_hbm.at[idx_v.at[j]], bufs[bf], gsem[bf])

        def wait_g(bf):
            pltpu.make_async_copy(table_hbm.at[idx_v.at[0]], bufs[bf],
                                  gsem[bf]).wait()

        def start_o(j, bf):
            pltpu.async_copy(bufs[bf], out_hbm.at[base + j], osem[bf])

        def wait_o(bf):
            pltpu.make_async_copy(bufs[bf], out_hbm.at[base], osem[bf]).wait()

        start_g(0, 0)
        wait_g(0)
        start_g(1, 1)
        start_o(0, 0)

        def pair(gi, carry):
            j = 2 * gi + 1
            wait_g(1)
            wait_o(0)
            start_g(j + 1, 0)
            start_o(j, 1)
            wait_g(0)
            wait_o(1)
            start_g(j + 2, 1)
            start_o(j + 1, 0)
            return carry

        lax.fori_loop(0, (nchunk - 2) // 2, pair, 0)

        wait_g(1)
        wait_o(0)
        start_o(nchunk - 1, 1)
        wait_o(1)

    return gather_kernel, nw, nchunk


def kernel(x, table):
    b, l = x.shape
    d = table.shape[1]
    gather, nw, nchunk = _make_gather(b, l, d)
    idx = x.astype(jnp.int32).reshape(nw, nchunk, l)
    return gather(idx, table)


# speedup vs baseline: 2.1000x; 2.0109x over previous
import functools

import jax
import jax.numpy as jnp
from jax import lax
from jax.experimental import pallas as pl
from jax.experimental.pallas import tpu as pltpu
from jax.experimental.pallas import tpu_sc as plsc


@functools.lru_cache(maxsize=None)
def _make_gather(b: int, l: int, dp: int):
    info = plsc.get_sparse_core_info()
    nc, ns = info.num_cores, info.num_subcores
    nw = nc * ns
    assert b % nw == 0
    nchunk = b // nw
    assert nchunk % 2 == 0
    mesh = plsc.VectorSubcoreMesh(core_axis_name="c", subcore_axis_name="s")

    @functools.partial(
        pl.kernel,
        out_type=jax.ShapeDtypeStruct((b, l, dp), jnp.float32),
        mesh=mesh,
        scratch_types=[
            pltpu.VMEM((nchunk, l), jnp.int32),
            pltpu.VMEM((l, dp), jnp.float32),
            pltpu.VMEM((l, dp), jnp.float32),
            pltpu.SemaphoreType.DMA,
            pltpu.SemaphoreType.DMA,
            pltpu.SemaphoreType.DMA,
            pltpu.SemaphoreType.DMA,
        ],
        compiler_params=pltpu.CompilerParams(use_tc_tiling_on_sc=True),
    )
    def gather_kernel(idx_hbm, table_hbm, out_hbm,
                      idx_v, buf0, buf1, g0, g1, o0, o1):
        wid = lax.axis_index("s") * nc + lax.axis_index("c")
        base = wid * nchunk
        pltpu.sync_copy(idx_hbm.at[wid], idx_v)

        bufs = (buf0, buf1)
        gsem = (g0, g1)
        osem = (o0, o1)

        def start_g(j, bf):
            pltpu.async_copy(table_hbm.at[idx_v.at[j]], bufs[bf], gsem[bf])

        def wait_g(bf):
            pltpu.make_async_copy(table_hbm.at[idx_v.at[0]], bufs[bf],
                                  gsem[bf]).wait()

        def start_o(j, bf):
            pltpu.async_copy(bufs[bf], out_hbm.at[base + j], osem[bf])

        def wait_o(bf):
            pltpu.make_async_copy(bufs[bf], out_hbm.at[base], osem[bf]).wait()

        start_g(0, 0)
        wait_g(0)
        start_g(1, 1)
        start_o(0, 0)

        def pair(gi, carry):
            j = 2 * gi + 1
            wait_g(1)
            wait_o(0)
            start_g(j + 1, 0)
            start_o(j, 1)
            wait_g(0)
            wait_o(1)
            start_g(j + 2, 1)
            start_o(j + 1, 0)
            return carry

        lax.fori_loop(0, (nchunk - 2) // 2, pair, 0)

        wait_g(1)
        wait_o(0)
        start_o(nchunk - 1, 1)
        wait_o(1)

    return gather_kernel, nw, nchunk


def kernel(x, table):
    b, l = x.shape
    d = table.shape[1]
    dp = (d + 127) // 128 * 128
    gather, nw, nchunk = _make_gather(b, l, dp)
    idx = x.astype(jnp.int32).reshape(nw, nchunk, l)
    table_p = jnp.pad(table, ((0, 0), (0, dp - d)))
    out = gather(idx, table_p)
    return out[:, :, :d]
